# baseline (speedup 1.0000x reference)
import functools

import jax
import jax.numpy as jnp
from jax.experimental import pallas as pl
from jax.experimental.pallas import tpu as pltpu

EPS = 1e-5
LANE = 128

_CP = pltpu.CompilerParams(
    dimension_semantics=("parallel",),
    vmem_limit_bytes=64 * 1024 * 1024,
)


def _rup(x, m):
    return (x + m - 1) // m * m


def _tile(m, target):
    if m <= target:
        return m
    for t in range(target, 7, -1):
        if m % t == 0 and t % 8 == 0:
            return t
    return m


def _pad_last(a, c):
    pad = c - a.shape[-1]
    if pad == 0:
        return a
    return jnp.pad(a, [(0, 0)] * (a.ndim - 1) + [(0, pad)])


def _stats(yf):
    return jnp.concatenate(
        [jnp.sum(yf, 0, keepdims=True), jnp.sum(yf * yf, 0, keepdims=True)], 0)


def _stats3(yf):
    return jnp.stack([jnp.sum(yf, axis=(0, 1)),
                      jnp.sum(yf * yf, axis=(0, 1))])


def _fold2(s, ss, count, gamma, beta):
    mean = s / count
    var = jnp.maximum(ss / count - mean * mean, 0.0)
    scale = gamma * jax.lax.rsqrt(var + EPS)
    return scale, beta - mean * scale


def _fold(st, count, gamma, beta):
    s = jnp.sum(st[:, 0, :], axis=0, keepdims=True)
    ss = jnp.sum(st[:, 1, :], axis=0, keepdims=True)
    return _fold2(s, ss, count, gamma, beta)


def _dot2(a3, w):
    return jax.lax.dot_general(
        a3, w, dimension_numbers=(((2,), (0,)), ((), ())),
        preferred_element_type=jnp.float32)


def _mm_stats_kernel(x_ref, w_ref, y_ref, st_ref):
    y = jnp.dot(x_ref[...], w_ref[...], preferred_element_type=jnp.float32)
    y_ref[...] = y
    st_ref[0] = _stats(y)


def _conv2_kernel(y1_ref, st1_ref, g1_ref, c1_ref, w2_ref, y2_ref, st_ref,
                  z_ref, *, ho, wo, m1):
    cp = y1_ref.shape[-1]
    s1, b1 = _fold(st1_ref[...], m1, g1_ref[...], c1_ref[...])
    s1 = s1.reshape(1, 1, cp)
    b1 = b1.reshape(1, 1, cp)
    for a in range(2):
        for b in range(2):
            yp = y1_ref[0, pl.Slice(a, ho, 2), pl.Slice(b, wo, 2), :]
            z_ref[a * 2 + b] = jnp.maximum(
                yp * s1 + b1, 0.0).astype(jnp.bfloat16)
    acc = jnp.zeros((ho, wo, w2_ref.shape[-1]), jnp.float32)
    for ky in range(3):
        parts = []
        for kx in range(3):
            a, b = (ky + 1) % 2, (kx + 1) % 2
            base = z_ref[a * 2 + b]
            if ky == 0:
                base = jnp.concatenate(
                    [jnp.zeros((1, wo, cp), base.dtype), base[:ho - 1]], 0)
            if kx == 0:
                base = jnp.concatenate(
                    [jnp.zeros((ho, 1, cp), base.dtype), base[:, :wo - 1]], 1)
            parts.append(base)
        wide = jnp.concatenate(parts, axis=2)
        acc = acc + _dot2(wide, w2_ref[ky])
    yb = acc.astype(jnp.bfloat16)
    y2_ref[0] = yb
    st_ref[0] = _stats3(yb.astype(jnp.float32))


def _dual_mm_kernel(y2_ref, st2_ref, g2_ref, c2_ref, w3_ref, xs_ref,
                    wse_ref, wso_ref, y3_ref, st3_ref, ysc_ref, stsc_ref,
                    *, wo, m2):
    cp = y2_ref.shape[-1]
    s2, b2 = _fold(st2_ref[...], m2, g2_ref[...], c2_ref[...])
    z2 = jnp.maximum(
        y2_ref[0].astype(jnp.float32) * s2.reshape(1, 1, cp)
        + b2.reshape(1, 1, cp), 0.0).astype(jnp.bfloat16)
    y3 = _dot2(z2, w3_ref[...]).astype(jnp.bfloat16)
    y3_ref[0] = y3
    st3_ref[0] = _stats3(y3.astype(jnp.float32))
    xu = xs_ref[0, :, pl.Slice(0, wo, 2), :]
    fe = jax.lax.bitcast_convert_type(xu << 16, jnp.float32)
    fo = jax.lax.bitcast_convert_type(xu & jnp.uint32(0xFFFF0000),
                                      jnp.float32)
    ysc = (_dot2(fe.astype(jnp.bfloat16), wse_ref[...])
           + _dot2(fo.astype(jnp.bfloat16), wso_ref[...])
           ).astype(jnp.bfloat16)
    ysc_ref[0] = ysc
    stsc_ref[0] = _stats3(ysc.astype(jnp.float32))


def _final_kernel(y3_ref, ysc_ref, st3_ref, g3_ref, c3_ref,
                  stsc_ref, gs_ref, cs_ref, o_ref, *, m2):
    cpo = y3_ref.shape[-1]
    s3, b3 = _fold(st3_ref[...], m2, g3_ref[...], c3_ref[...])
    ssc, bsc = _fold(stsc_ref[...], m2, gs_ref[...], cs_ref[...])
    a = y3_ref[0].astype(jnp.float32) * s3.reshape(1, 1, cpo) \
        + b3.reshape(1, 1, cpo)
    b = ysc_ref[0].astype(jnp.float32) * ssc.reshape(1, 1, cpo) \
        + bsc.reshape(1, 1, cpo)
    o_ref[0] = jnp.maximum(a + b, 0.0)


def kernel(x, w1, g1, b1, w2, g2, b2, w3, g3, b3, ws, gs, bs):
    N, Cin, H, W = x.shape
    planes = w1.shape[0]
    cout = w3.shape[0]
    cp = _rup(planes, LANE)
    cpo = _rup(cout, LANE)
    Ho, Wo = (H - 1) // 2 + 1, (W - 1) // 2 + 1
    hw = Ho * Wo
    M1, M2 = N * H * W, N * hw
    bf = jnp.bfloat16
    f32 = jnp.float32

    w1m = _pad_last(w1[:, :, 0, 0].T, cp).astype(bf)
    w2t = jnp.transpose(w2, (2, 3, 1, 0))
    w2m = jnp.pad(
        w2t, ((0, 0), (0, 0), (0, cp - planes), (0, cp - planes))
    ).reshape(3, 3 * cp, cp).astype(bf)
    w3m = jnp.pad(
        w3[:, :, 0, 0].T, ((0, cp - planes), (0, cpo - cout))).astype(bf)
    wsm = _pad_last(ws[:, :, 0, 0].T, cpo).astype(bf)
    g1p, b1p = _pad_last(g1, cp).reshape(1, cp), _pad_last(b1, cp).reshape(1, cp)
    g2p, b2p = _pad_last(g2, cp).reshape(1, cp), _pad_last(b2, cp).reshape(1, cp)
    g3p, b3p = (_pad_last(g3, cpo).reshape(1, cpo),
                _pad_last(b3, cpo).reshape(1, cpo))
    gsp, bsp = (_pad_last(gs, cpo).reshape(1, cpo),
                _pad_last(bs, cpo).reshape(1, cpo))

    xt = jnp.transpose(x, (0, 2, 3, 1)).astype(bf)
    x2d = xt.reshape(M1, Cin)
    xtu = jax.lax.bitcast_convert_type(
        xt.reshape(N, Ho, 2, W, Cin // 2, 2), jnp.uint32
    )
    wse = wsm[0::2, :]
    wso = wsm[1::2, :]

    TM1 = _tile(M1, 4 * hw)
    gr1 = M1 // TM1
    y1, st1 = pl.pallas_call(
        _mm_stats_kernel,
        grid=(gr1,),
        in_specs=[pl.BlockSpec((TM1, Cin), lambda i: (i, 0)),
                  pl.BlockSpec((Cin, cp), lambda i: (0, 0))],
        out_specs=[pl.BlockSpec((TM1, cp), lambda i: (i, 0)),
                   pl.BlockSpec((1, 2, cp), lambda i: (i, 0, 0))],
        out_shape=[jax.ShapeDtypeStruct((M1, cp), f32),
                   jax.ShapeDtypeStruct((gr1, 2, cp), f32)],
        compiler_params=_CP,
    )(x2d, w1m)

    k2 = functools.partial(_conv2_kernel, ho=Ho, wo=Wo, m1=float(M1))
    y2, st2 = pl.pallas_call(
        k2,
        grid=(N,),
        in_specs=[pl.BlockSpec((1, H, W, cp), lambda n: (n, 0, 0, 0)),
                  pl.BlockSpec((gr1, 2, cp), lambda n: (0, 0, 0)),
                  pl.BlockSpec((1, cp), lambda n: (0, 0)),
                  pl.BlockSpec((1, cp), lambda n: (0, 0)),
                  pl.BlockSpec((3, 3 * cp, cp), lambda n: (0, 0, 0))],
        out_specs=[pl.BlockSpec((1, Ho, Wo, cp), lambda n: (n, 0, 0, 0)),
                   pl.BlockSpec((1, 2, cp), lambda n: (n, 0, 0))],
        out_shape=[jax.ShapeDtypeStruct((N, Ho, Wo, cp), bf),
                   jax.ShapeDtypeStruct((N, 2, cp), f32)],
        scratch_shapes=[pltpu.VMEM((4, Ho, Wo, cp), bf)],
        compiler_params=_CP,
    )(y1.reshape(N, H, W, cp), st1, g1p, b1p, w2m)

    k3 = functools.partial(_dual_mm_kernel, wo=Wo, m2=float(M2))
    y3, st3, ysc, stsc = pl.pallas_call(
        k3,
        grid=(N,),
        in_specs=[pl.BlockSpec((1, Ho, Wo, cp), lambda n: (n, 0, 0, 0)),
                  pl.BlockSpec((N, 2, cp), lambda n: (0, 0, 0)),
                  pl.BlockSpec((1, cp), lambda n: (0, 0)),
                  pl.BlockSpec((1, cp), lambda n: (0, 0)),
                  pl.BlockSpec((cp, cpo), lambda n: (0, 0)),
                  pl.BlockSpec((1, Ho, None, W, Cin // 2),
                               lambda n: (n, 0, 0, 0, 0)),
                  pl.BlockSpec((Cin // 2, cpo), lambda n: (0, 0)),
                  pl.BlockSpec((Cin // 2, cpo), lambda n: (0, 0))],
        out_specs=[pl.BlockSpec((1, Ho, Wo, cpo), lambda n: (n, 0, 0, 0)),
                   pl.BlockSpec((1, 2, cpo), lambda n: (n, 0, 0)),
                   pl.BlockSpec((1, Ho, Wo, cpo), lambda n: (n, 0, 0, 0)),
                   pl.BlockSpec((1, 2, cpo), lambda n: (n, 0, 0))],
        out_shape=[jax.ShapeDtypeStruct((N, Ho, Wo, cpo), bf),
                   jax.ShapeDtypeStruct((N, 2, cpo), f32),
                   jax.ShapeDtypeStruct((N, Ho, Wo, cpo), bf),
                   jax.ShapeDtypeStruct((N, 2, cpo), f32)],
        compiler_params=_CP,
    )(y2, st2, g2p, b2p, w3m, xtu, wse, wso)

    k4 = functools.partial(_final_kernel, m2=float(M2))
    out = pl.pallas_call(
        k4,
        grid=(N,),
        in_specs=[pl.BlockSpec((1, Ho, Wo, cpo), lambda n: (n, 0, 0, 0)),
                  pl.BlockSpec((1, Ho, Wo, cpo), lambda n: (n, 0, 0, 0)),
                  pl.BlockSpec((N, 2, cpo), lambda n: (0, 0, 0)),
                  pl.BlockSpec((1, cpo), lambda n: (0, 0)),
                  pl.BlockSpec((1, cpo), lambda n: (0, 0)),
                  pl.BlockSpec((N, 2, cpo), lambda n: (0, 0, 0)),
                  pl.BlockSpec((1, cpo), lambda n: (0, 0)),
                  pl.BlockSpec((1, cpo), lambda n: (0, 0))],
        out_specs=pl.BlockSpec((1, Ho, Wo, cpo), lambda n: (n, 0, 0, 0)),
        out_shape=jax.ShapeDtypeStruct((N, Ho, Wo, cpo), f32),
        compiler_params=_CP,
    )(y3, ysc, st3, g3p, b3p, stsc, gsp, bsp)

    return jnp.transpose(out[..., :cout], (0, 3, 1, 2))


# speedup vs baseline: 1.6089x; 1.6089x over previous
import functools

import jax
import jax.numpy as jnp
from jax.experimental import pallas as pl
from jax.experimental.pallas import tpu as pltpu

EPS = 1e-5
LANE = 128

_CP = pltpu.CompilerParams(
    dimension_semantics=("parallel",),
    vmem_limit_bytes=64 * 1024 * 1024,
)


def _rup(x, m):
    return (x + m - 1) // m * m


def _tile(m, target):
    if m <= target:
        return m
    for t in range(target, 7, -1):
        if m % t == 0 and t % 8 == 0:
            return t
    return m


def _pad_last(a, c):
    pad = c - a.shape[-1]
    if pad == 0:
        return a
    return jnp.pad(a, [(0, 0)] * (a.ndim - 1) + [(0, pad)])


def _stats(yf):
    return jnp.concatenate(
        [jnp.sum(yf, 0, keepdims=True), jnp.sum(yf * yf, 0, keepdims=True)], 0)


def _stats3(yf):
    return jnp.stack([jnp.sum(yf, axis=(0, 1)),
                      jnp.sum(yf * yf, axis=(0, 1))])


def _fold2(s, ss, count, gamma, beta):
    mean = s / count
    var = jnp.maximum(ss / count - mean * mean, 0.0)
    scale = gamma * jax.lax.rsqrt(var + EPS)
    return scale, beta - mean * scale


def _fold(st, count, gamma, beta):
    s = jnp.sum(st[:, 0, :], axis=0, keepdims=True)
    ss = jnp.sum(st[:, 1, :], axis=0, keepdims=True)
    return _fold2(s, ss, count, gamma, beta)


def _dot2(a3, w):
    return jax.lax.dot_general(
        a3, w, dimension_numbers=(((2,), (0,)), ((), ())),
        preferred_element_type=jnp.float32)


def _mm_stats_kernel(x_ref, w_ref, y_ref, st_ref):
    y = jnp.dot(x_ref[...], w_ref[...], preferred_element_type=jnp.float32)
    y_ref[...] = y
    st_ref[0] = _stats(y)


def _conv2_kernel(y1_ref, st1_ref, g1_ref, c1_ref, w2_ref, y2_ref, st_ref,
                  z_ref, *, ho, wo, m1):
    cp = y1_ref.shape[-1]
    s1, b1 = _fold(st1_ref[...], m1, g1_ref[...], c1_ref[...])
    s1 = s1.reshape(1, 1, cp)
    b1 = b1.reshape(1, 1, cp)
    for a in range(2):
        for b in range(2):
            yp = y1_ref[0, pl.Slice(a, ho, 2), pl.Slice(b, wo, 2), :]
            z_ref[a * 2 + b] = jnp.maximum(
                yp * s1 + b1, 0.0).astype(jnp.bfloat16)
    acc = jnp.zeros((ho, wo, w2_ref.shape[-1]), jnp.float32)
    for ky in range(3):
        parts = []
        for kx in range(3):
            a, b = (ky + 1) % 2, (kx + 1) % 2
            base = z_ref[a * 2 + b]
            if ky == 0:
                base = jnp.concatenate(
                    [jnp.zeros((1, wo, cp), base.dtype), base[:ho - 1]], 0)
            if kx == 0:
                base = jnp.concatenate(
                    [jnp.zeros((ho, 1, cp), base.dtype), base[:, :wo - 1]], 1)
            parts.append(base)
        wide = jnp.concatenate(parts, axis=2)
        acc = acc + _dot2(wide, w2_ref[ky])
    yb = acc.astype(jnp.bfloat16)
    y2_ref[0] = yb
    st_ref[0] = _stats3(yb.astype(jnp.float32))


def _dual_mm_kernel(y2_ref, st2_ref, g2_ref, c2_ref, w3_ref, xs_ref, ws_ref,
                    y3_ref, st3_ref, ysc_ref, stsc_ref, *, m2):
    cp = y2_ref.shape[-1]
    s2, b2 = _fold(st2_ref[...], m2, g2_ref[...], c2_ref[...])
    z2 = jnp.maximum(
        y2_ref[0].astype(jnp.float32) * s2.reshape(1, 1, cp)
        + b2.reshape(1, 1, cp), 0.0).astype(jnp.bfloat16)
    y3 = _dot2(z2, w3_ref[...]).astype(jnp.bfloat16)
    y3_ref[0] = y3
    st3_ref[0] = _stats3(y3.astype(jnp.float32))
    ysc = _dot2(xs_ref[0], ws_ref[...]).astype(jnp.bfloat16)
    ysc_ref[0] = ysc
    stsc_ref[0] = _stats3(ysc.astype(jnp.float32))


def _final_kernel(y3_ref, ysc_ref, st3_ref, g3_ref, c3_ref,
                  stsc_ref, gs_ref, cs_ref, o_ref, *, m2):
    cpo = y3_ref.shape[-1]
    s3, b3 = _fold(st3_ref[...], m2, g3_ref[...], c3_ref[...])
    ssc, bsc = _fold(stsc_ref[...], m2, gs_ref[...], cs_ref[...])
    a = y3_ref[0].astype(jnp.float32) * s3.reshape(1, 1, cpo) \
        + b3.reshape(1, 1, cpo)
    b = ysc_ref[0].astype(jnp.float32) * ssc.reshape(1, 1, cpo) \
        + bsc.reshape(1, 1, cpo)
    o_ref[0] = jnp.maximum(a + b, 0.0)


def kernel(x, w1, g1, b1, w2, g2, b2, w3, g3, b3, ws, gs, bs):
    N, Cin, H, W = x.shape
    planes = w1.shape[0]
    cout = w3.shape[0]
    cp = _rup(planes, LANE)
    cpo = _rup(cout, LANE)
    Ho, Wo = (H - 1) // 2 + 1, (W - 1) // 2 + 1
    hw = Ho * Wo
    M1, M2 = N * H * W, N * hw
    bf = jnp.bfloat16
    f32 = jnp.float32

    w1m = _pad_last(w1[:, :, 0, 0].T, cp).astype(bf)
    w2t = jnp.transpose(w2, (2, 3, 1, 0))
    w2m = jnp.pad(
        w2t, ((0, 0), (0, 0), (0, cp - planes), (0, cp - planes))
    ).reshape(3, 3 * cp, cp).astype(bf)
    w3m = jnp.pad(
        w3[:, :, 0, 0].T, ((0, cp - planes), (0, cpo - cout))).astype(bf)
    wsm = _pad_last(ws[:, :, 0, 0].T, cpo).astype(bf)
    g1p, b1p = _pad_last(g1, cp).reshape(1, cp), _pad_last(b1, cp).reshape(1, cp)
    g2p, b2p = _pad_last(g2, cp).reshape(1, cp), _pad_last(b2, cp).reshape(1, cp)
    g3p, b3p = (_pad_last(g3, cpo).reshape(1, cpo),
                _pad_last(b3, cpo).reshape(1, cpo))
    gsp, bsp = (_pad_last(gs, cpo).reshape(1, cpo),
                _pad_last(bs, cpo).reshape(1, cpo))

    xt = jnp.transpose(x, (0, 2, 3, 1)).astype(bf)
    x2d = xt.reshape(M1, Cin)
    xs4 = xt.reshape(N, Ho, 2, Wo, 2, Cin)[:, :, 0, :, 0, :]

    TM1 = _tile(M1, 4 * hw)
    gr1 = M1 // TM1
    y1, st1 = pl.pallas_call(
        _mm_stats_kernel,
        grid=(gr1,),
        in_specs=[pl.BlockSpec((TM1, Cin), lambda i: (i, 0)),
                  pl.BlockSpec((Cin, cp), lambda i: (0, 0))],
        out_specs=[pl.BlockSpec((TM1, cp), lambda i: (i, 0)),
                   pl.BlockSpec((1, 2, cp), lambda i: (i, 0, 0))],
        out_shape=[jax.ShapeDtypeStruct((M1, cp), f32),
                   jax.ShapeDtypeStruct((gr1, 2, cp), f32)],
        compiler_params=_CP,
    )(x2d, w1m)

    k2 = functools.partial(_conv2_kernel, ho=Ho, wo=Wo, m1=float(M1))
    y2, st2 = pl.pallas_call(
        k2,
        grid=(N,),
        in_specs=[pl.BlockSpec((1, H, W, cp), lambda n: (n, 0, 0, 0)),
                  pl.BlockSpec((gr1, 2, cp), lambda n: (0, 0, 0)),
                  pl.BlockSpec((1, cp), lambda n: (0, 0)),
                  pl.BlockSpec((1, cp), lambda n: (0, 0)),
                  pl.BlockSpec((3, 3 * cp, cp), lambda n: (0, 0, 0))],
        out_specs=[pl.BlockSpec((1, Ho, Wo, cp), lambda n: (n, 0, 0, 0)),
                   pl.BlockSpec((1, 2, cp), lambda n: (n, 0, 0))],
        out_shape=[jax.ShapeDtypeStruct((N, Ho, Wo, cp), bf),
                   jax.ShapeDtypeStruct((N, 2, cp), f32)],
        scratch_shapes=[pltpu.VMEM((4, Ho, Wo, cp), bf)],
        compiler_params=_CP,
    )(y1.reshape(N, H, W, cp), st1, g1p, b1p, w2m)

    k3 = functools.partial(_dual_mm_kernel, m2=float(M2))
    y3, st3, ysc, stsc = pl.pallas_call(
        k3,
        grid=(N,),
        in_specs=[pl.BlockSpec((1, Ho, Wo, cp), lambda n: (n, 0, 0, 0)),
                  pl.BlockSpec((N, 2, cp), lambda n: (0, 0, 0)),
                  pl.BlockSpec((1, cp), lambda n: (0, 0)),
                  pl.BlockSpec((1, cp), lambda n: (0, 0)),
                  pl.BlockSpec((cp, cpo), lambda n: (0, 0)),
                  pl.BlockSpec((1, Ho, Wo, Cin), lambda n: (n, 0, 0, 0)),
                  pl.BlockSpec((Cin, cpo), lambda n: (0, 0))],
        out_specs=[pl.BlockSpec((1, Ho, Wo, cpo), lambda n: (n, 0, 0, 0)),
                   pl.BlockSpec((1, 2, cpo), lambda n: (n, 0, 0)),
                   pl.BlockSpec((1, Ho, Wo, cpo), lambda n: (n, 0, 0, 0)),
                   pl.BlockSpec((1, 2, cpo), lambda n: (n, 0, 0))],
        out_shape=[jax.ShapeDtypeStruct((N, Ho, Wo, cpo), bf),
                   jax.ShapeDtypeStruct((N, 2, cpo), f32),
                   jax.ShapeDtypeStruct((N, Ho, Wo, cpo), bf),
                   jax.ShapeDtypeStruct((N, 2, cpo), f32)],
        compiler_params=_CP,
    )(y2, st2, g2p, b2p, w3m, xs4, wsm)

    k4 = functools.partial(_final_kernel, m2=float(M2))
    out = pl.pallas_call(
        k4,
        grid=(N,),
        in_specs=[pl.BlockSpec((1, Ho, Wo, cpo), lambda n: (n, 0, 0, 0)),
                  pl.BlockSpec((1, Ho, Wo, cpo), lambda n: (n, 0, 0, 0)),
                  pl.BlockSpec((N, 2, cpo), lambda n: (0, 0, 0)),
                  pl.BlockSpec((1, cpo), lambda n: (0, 0)),
                  pl.BlockSpec((1, cpo), lambda n: (0, 0)),
                  pl.BlockSpec((N, 2, cpo), lambda n: (0, 0, 0)),
                  pl.BlockSpec((1, cpo), lambda n: (0, 0)),
                  pl.BlockSpec((1, cpo), lambda n: (0, 0))],
        out_specs=pl.BlockSpec((1, Ho, Wo, cpo), lambda n: (n, 0, 0, 0)),
        out_shape=jax.ShapeDtypeStruct((N, Ho, Wo, cpo), f32),
        compiler_params=_CP,
    )(y3, ysc, st3, g3p, b3p, stsc, gsp, bsp)

    return jnp.transpose(out[..., :cout], (0, 3, 1, 2))
